# baseline (speedup 1.0000x reference)
import math

import jax
import jax.numpy as jnp
import numpy as np
from jax import lax
from jax.experimental import pallas as pl
from jax.experimental.pallas import tpu as pltpu
from jax.experimental.pallas import tpu_sc as plsc

_ROWS = 512
_I0 = np.int32(0)
_EPS = np.float32(1e-7)


def _scores_body(temp_ref, src_ref, tgt_ref, w_ref, idx_ref):
    b = pl.program_id(0)
    n = tgt_ref.shape[2]
    t = temp_ref[b, 0, 0]
    a = src_ref[0]
    bt = tgt_ref[0]
    scale = np.float32(math.sqrt(src_ref.shape[1]))
    s = lax.dot_general(a, bt, (((0,), (0,)), ((), ())),
                        preferred_element_type=jnp.float32)
    z = t * (s * np.float32(1.0 / float(scale)))
    m = jnp.max(z, axis=1, keepdims=True)
    e = jnp.exp(z - m)
    ssum = jnp.sum(e, axis=1, keepdims=True)
    w = np.float32(1.0) / ssum
    ii = lax.broadcasted_iota(jnp.int32, e.shape, 1)
    cand = jnp.where(e == np.float32(1.0), ii, jnp.int32(n))
    idx = jnp.min(cand, axis=1)
    w_ref[0, 0] = w[:, 0]
    idx_ref[0, 0] = idx


def _max_argmax(src_embedding, tgt_embedding, temperature):
    bb, d, n = src_embedding.shape
    nt = n // _ROWS
    grid = (bb, nt)
    w3, i3 = pl.pallas_call(
        _scores_body,
        grid=grid,
        in_specs=[
            pl.BlockSpec(memory_space=pltpu.SMEM),
            pl.BlockSpec((1, d, _ROWS), lambda b, r: (b, b * 0, r)),
            pl.BlockSpec((1, d, n), lambda b, r: (b, b * 0, b * 0)),
        ],
        out_specs=[
            pl.BlockSpec((1, 1, _ROWS), lambda b, r: (b * nt + r, b * 0, b * 0)),
            pl.BlockSpec((1, 1, _ROWS), lambda b, r: (b * nt + r, b * 0, b * 0)),
        ],
        out_shape=[
            jax.ShapeDtypeStruct((bb * nt, 1, _ROWS), jnp.float32),
            jax.ShapeDtypeStruct((bb * nt, 1, _ROWS), jnp.int32),
        ],
    )(temperature, src_embedding, tgt_embedding)
    return w3.reshape(bb, n), i3.reshape(bb, n)


def _gather_body(tgt_ref, cor_ref, y_ref, t0, t1, t2, idxv, y0, y1, y2):
    i32 = jnp.int32
    cid = lax.axis_index("c").astype(i32)
    sid = lax.axis_index("s").astype(i32)
    wid = sid * i32(2) + cid
    b = lax.shift_right_logical(wid, i32(2))
    q = lax.bitwise_and(wid, i32(3))
    chunk = 512
    start = q * i32(chunk)
    row = b * i32(3)
    pltpu.sync_copy(tgt_ref.at[row], t0)
    pltpu.sync_copy(tgt_ref.at[row + i32(1)], t1)
    pltpu.sync_copy(tgt_ref.at[row + i32(2)], t2)
    pltpu.sync_copy(cor_ref.at[b, pl.ds(start, chunk)], idxv)

    for i in range(chunk // 16):
        sl = pl.ds(i * 16, 16)
        idx16 = idxv[sl]
        y0[sl] = plsc.load_gather(t0, [idx16])
        y1[sl] = plsc.load_gather(t1, [idx16])
        y2[sl] = plsc.load_gather(t2, [idx16])
    pltpu.sync_copy(y0, y_ref.at[row, pl.ds(start, chunk)])
    pltpu.sync_copy(y1, y_ref.at[row + i32(1), pl.ds(start, chunk)])
    pltpu.sync_copy(y2, y_ref.at[row + i32(2), pl.ds(start, chunk)])


def _sc_gather(tgt, corres):
    bb, _, n = tgt.shape
    chunk = 512
    mesh = plsc.VectorSubcoreMesh(core_axis_name="c", subcore_axis_name="s")
    y2d = pl.kernel(
        _gather_body,
        out_type=jax.ShapeDtypeStruct((bb * 3, n), jnp.float32),
        mesh=mesh,
        compiler_params=pltpu.CompilerParams(needs_layout_passes=False),
        scratch_types=[
            pltpu.VMEM((n,), jnp.float32),
            pltpu.VMEM((n,), jnp.float32),
            pltpu.VMEM((n,), jnp.float32),
            pltpu.VMEM((chunk,), jnp.int32),
            pltpu.VMEM((chunk,), jnp.float32),
            pltpu.VMEM((chunk,), jnp.float32),
            pltpu.VMEM((chunk,), jnp.float32),
        ],
    )(tgt.reshape(bb * 3, n), corres)
    return y2d.reshape(bb, 3, n)


def _finish_body(w_ref, x_ref, y_ref, r_ref, t_ref):
    w = w_ref[...]
    x = x_ref[...]
    y = y_ref[...]
    bb = w.shape[0]

    tot = jnp.sum(jnp.abs(w), axis=1)
    pp = tot + _EPS
    wb = w[:, None, :]
    sx = jnp.sum(x * wb, axis=2)
    sy = jnp.sum(y * wb, axis=2)
    mx = sx / pp[:, None]
    my = sy / pp[:, None]
    corr = (np.float32(2.0) - tot / pp)[:, None]

    bcol = []
    for c in range(3):
        wx = (w * x[:, c, :])[:, None, :]
        syx_c = jnp.sum(y * wx, axis=2)
        bcol.append(syx_c / pp[:, None] - my * (mx[:, c:c + 1] * corr))

    det = (
        bcol[0][:, 0] * (bcol[1][:, 1] * bcol[2][:, 2] - bcol[1][:, 2] * bcol[2][:, 1])
        - bcol[1][:, 0] * (bcol[0][:, 1] * bcol[2][:, 2] - bcol[0][:, 2] * bcol[2][:, 1])
        + bcol[2][:, 0] * (bcol[0][:, 1] * bcol[1][:, 2] - bcol[0][:, 2] * bcol[1][:, 1])
    )

    lane = lax.broadcasted_iota(jnp.int32, (bb, 3), 1)
    vcol = [(lane == k).astype(jnp.float32) for k in range(3)]

    one = np.float32(1.0)
    for _ in range(7):
        for (ip, iq) in ((0, 1), (0, 2), (1, 2)):
            bp, bq = bcol[ip], bcol[iq]
            alpha = jnp.sum(bp * bp, axis=1)
            beta = jnp.sum(bq * bq, axis=1)
            gamma = jnp.sum(bp * bq, axis=1)
            g0 = gamma == 0.0
            gsafe = jnp.where(g0, one, gamma)
            tau = (beta - alpha) / (np.float32(2.0) * gsafe)
            rr = jnp.sqrt(one + tau * tau)
            tt = jnp.where(tau >= 0.0, one / (tau + rr), one / (tau - rr))
            cc = one / jnp.sqrt(one + tt * tt)
            ss = cc * tt
            cc = jnp.where(g0, one, cc)[:, None]
            ss = jnp.where(g0, np.float32(0.0), ss)[:, None]
            newp = cc * bp - ss * bq
            newq = ss * bp + cc * bq
            bcol[ip], bcol[iq] = newp, newq
            vp, vq = vcol[ip], vcol[iq]
            vcol[ip] = cc * vp - ss * vq
            vcol[iq] = ss * vp + cc * vq

    sig = [jnp.sqrt(jnp.sum(bcol[k] * bcol[k], axis=1)) for k in range(3)]
    is0 = (sig[0] <= sig[1]) & (sig[0] <= sig[2])
    is1 = (~is0) & (sig[1] <= sig[2])
    is2 = ~(is0 | is1)
    neg = det < 0.0
    flips = [is0 & neg, is1 & neg, is2 & neg]

    r = jnp.zeros((bb, 3, 3), jnp.float32)
    for k in range(3):
        sgn = jnp.where(flips[k], np.float32(-1.0), one)
        uk = bcol[k] / jnp.maximum(sig[k], np.float32(1e-30))[:, None]
        r = r + (sgn[:, None, None] * uk[:, :, None]) * vcol[k][:, None, :]

    rmx = jnp.sum(r * mx[:, None, :], axis=2)
    t_out = my[:, None, :] - rmx[:, :, None]
    r_ref[...] = r
    t_ref[...] = t_out


def _procrustes(weight, src, y):
    bb, n = weight.shape
    return pl.pallas_call(
        _finish_body,
        grid=(1,),
        in_specs=[
            pl.BlockSpec((bb, n), lambda i: (i * 0, i * 0)),
            pl.BlockSpec((bb, 3, n), lambda i: (i * 0, i * 0, i * 0)),
            pl.BlockSpec((bb, 3, n), lambda i: (i * 0, i * 0, i * 0)),
        ],
        out_specs=[
            pl.BlockSpec((bb, 3, 3), lambda i: (i * 0, i * 0, i * 0)),
            pl.BlockSpec((bb, 3, 3), lambda i: (i * 0, i * 0, i * 0)),
        ],
        out_shape=[
            jax.ShapeDtypeStruct((bb, 3, 3), jnp.float32),
            jax.ShapeDtypeStruct((bb, 3, 3), jnp.float32),
        ],
    )(weight, src, y)


def kernel(src_embedding, tgt_embedding, src, tgt, temperature, is_corr):
    with jax.enable_x64(False):
        weight, corres = _max_argmax(src_embedding, tgt_embedding, temperature)
        y = _sc_gather(tgt, corres)
        r, t = _procrustes(weight, src, y)
    return r, t, corres.astype(jnp.int64)[..., None], weight[..., None]


# speedup vs baseline: 1.1283x; 1.1283x over previous
import math

import jax
import jax.numpy as jnp
import numpy as np
from jax import lax
from jax.experimental import pallas as pl
from jax.experimental.pallas import tpu as pltpu
from jax.experimental.pallas import tpu_sc as plsc

_ROWS = 512
_I0 = np.int32(0)
_EPS = np.float32(1e-7)


def _scores_body(temp_ref, src_ref, tgt_ref, w_ref, idx_ref):
    b = pl.program_id(0)
    n = tgt_ref.shape[2]
    t = temp_ref[b, 0, 0]
    a = src_ref[0]
    bt = tgt_ref[0]
    scale = np.float32(math.sqrt(src_ref.shape[1]))
    s = lax.dot_general(a, bt, (((0,), (0,)), ((), ())),
                        preferred_element_type=jnp.float32)
    z = t * (s * np.float32(1.0 / float(scale)))
    m = jnp.max(z, axis=1, keepdims=True)
    e = jnp.exp(z - m)
    ssum = jnp.sum(e, axis=1, keepdims=True)
    w = np.float32(1.0) / ssum
    ii = lax.broadcasted_iota(jnp.int32, (1, n), 1).astype(jnp.float32)
    cand = jnp.where(e == np.float32(1.0), ii, np.float32(n))
    idx = jnp.min(cand, axis=1).astype(jnp.int32)
    w_ref[0, 0] = w[:, 0]
    idx_ref[0, 0] = idx


def _max_argmax(src_embedding, tgt_embedding, temperature):
    bb, d, n = src_embedding.shape
    nt = n // _ROWS
    grid = (bb, nt)
    w3, i3 = pl.pallas_call(
        _scores_body,
        grid=grid,
        in_specs=[
            pl.BlockSpec(memory_space=pltpu.SMEM),
            pl.BlockSpec((1, d, _ROWS), lambda b, r: (b, b * 0, r)),
            pl.BlockSpec((1, d, n), lambda b, r: (b, b * 0, b * 0)),
        ],
        out_specs=[
            pl.BlockSpec((1, 1, _ROWS), lambda b, r: (b * nt + r, b * 0, b * 0)),
            pl.BlockSpec((1, 1, _ROWS), lambda b, r: (b * nt + r, b * 0, b * 0)),
        ],
        out_shape=[
            jax.ShapeDtypeStruct((bb * nt, 1, _ROWS), jnp.float32),
            jax.ShapeDtypeStruct((bb * nt, 1, _ROWS), jnp.int32),
        ],
    )(temperature, src_embedding, tgt_embedding)
    return w3.reshape(bb, n), i3.reshape(bb, n)


def _gather_body(tgt_ref, cor_ref, y_ref, t0, t1, t2, idxv, y0, y1, y2):
    i32 = jnp.int32
    cid = lax.axis_index("c").astype(i32)
    sid = lax.axis_index("s").astype(i32)
    wid = sid * i32(2) + cid
    b = lax.shift_right_logical(wid, i32(2))
    q = lax.bitwise_and(wid, i32(3))
    chunk = 512
    start = q * i32(chunk)
    row = b * i32(3)
    pltpu.sync_copy(tgt_ref.at[row], t0)
    pltpu.sync_copy(tgt_ref.at[row + i32(1)], t1)
    pltpu.sync_copy(tgt_ref.at[row + i32(2)], t2)
    pltpu.sync_copy(cor_ref.at[b, pl.ds(start, chunk)], idxv)

    for i in range(chunk // 16):
        sl = pl.ds(i * 16, 16)
        idx16 = idxv[sl]
        y0[sl] = plsc.load_gather(t0, [idx16])
        y1[sl] = plsc.load_gather(t1, [idx16])
        y2[sl] = plsc.load_gather(t2, [idx16])
    pltpu.sync_copy(y0, y_ref.at[row, pl.ds(start, chunk)])
    pltpu.sync_copy(y1, y_ref.at[row + i32(1), pl.ds(start, chunk)])
    pltpu.sync_copy(y2, y_ref.at[row + i32(2), pl.ds(start, chunk)])


def _sc_gather(tgt, corres):
    bb, _, n = tgt.shape
    chunk = 512
    mesh = plsc.VectorSubcoreMesh(core_axis_name="c", subcore_axis_name="s")
    y2d = pl.kernel(
        _gather_body,
        out_type=jax.ShapeDtypeStruct((bb * 3, n), jnp.float32),
        mesh=mesh,
        compiler_params=pltpu.CompilerParams(needs_layout_passes=False),
        scratch_types=[
            pltpu.VMEM((n,), jnp.float32),
            pltpu.VMEM((n,), jnp.float32),
            pltpu.VMEM((n,), jnp.float32),
            pltpu.VMEM((chunk,), jnp.int32),
            pltpu.VMEM((chunk,), jnp.float32),
            pltpu.VMEM((chunk,), jnp.float32),
            pltpu.VMEM((chunk,), jnp.float32),
        ],
    )(tgt.reshape(bb * 3, n), corres)
    return y2d.reshape(bb, 3, n)


def _finish_body(w_ref, x_ref, y_ref, r_ref, t_ref):
    w = w_ref[...]
    x = x_ref[...]
    y = y_ref[...]
    bb = w.shape[0]

    tot = jnp.sum(jnp.abs(w), axis=1)
    pp = tot + _EPS
    wb = w[:, None, :]
    sx = jnp.sum(x * wb, axis=2)
    sy = jnp.sum(y * wb, axis=2)
    mx = sx / pp[:, None]
    my = sy / pp[:, None]
    corr = (np.float32(2.0) - tot / pp)[:, None]

    bcol = []
    for c in range(3):
        wx = (w * x[:, c, :])[:, None, :]
        syx_c = jnp.sum(y * wx, axis=2)
        bcol.append(syx_c / pp[:, None] - my * (mx[:, c:c + 1] * corr))

    det = (
        bcol[0][:, 0] * (bcol[1][:, 1] * bcol[2][:, 2] - bcol[1][:, 2] * bcol[2][:, 1])
        - bcol[1][:, 0] * (bcol[0][:, 1] * bcol[2][:, 2] - bcol[0][:, 2] * bcol[2][:, 1])
        + bcol[2][:, 0] * (bcol[0][:, 1] * bcol[1][:, 2] - bcol[0][:, 2] * bcol[1][:, 1])
    )

    lane = lax.broadcasted_iota(jnp.int32, (bb, 3), 1)
    vcol = [(lane == k).astype(jnp.float32) for k in range(3)]

    one = np.float32(1.0)
    for _ in range(7):
        for (ip, iq) in ((0, 1), (0, 2), (1, 2)):
            bp, bq = bcol[ip], bcol[iq]
            alpha = jnp.sum(bp * bp, axis=1)
            beta = jnp.sum(bq * bq, axis=1)
            gamma = jnp.sum(bp * bq, axis=1)
            g0 = gamma == 0.0
            gsafe = jnp.where(g0, one, gamma)
            tau = (beta - alpha) / (np.float32(2.0) * gsafe)
            rr = jnp.sqrt(one + tau * tau)
            tt = jnp.where(tau >= 0.0, one / (tau + rr), one / (tau - rr))
            cc = one / jnp.sqrt(one + tt * tt)
            ss = cc * tt
            cc = jnp.where(g0, one, cc)[:, None]
            ss = jnp.where(g0, np.float32(0.0), ss)[:, None]
            newp = cc * bp - ss * bq
            newq = ss * bp + cc * bq
            bcol[ip], bcol[iq] = newp, newq
            vp, vq = vcol[ip], vcol[iq]
            vcol[ip] = cc * vp - ss * vq
            vcol[iq] = ss * vp + cc * vq

    sig = [jnp.sqrt(jnp.sum(bcol[k] * bcol[k], axis=1)) for k in range(3)]
    is0 = (sig[0] <= sig[1]) & (sig[0] <= sig[2])
    is1 = (~is0) & (sig[1] <= sig[2])
    is2 = ~(is0 | is1)
    neg = det < 0.0
    flips = [is0 & neg, is1 & neg, is2 & neg]

    r = jnp.zeros((bb, 3, 3), jnp.float32)
    for k in range(3):
        sgn = jnp.where(flips[k], np.float32(-1.0), one)
        uk = bcol[k] / jnp.maximum(sig[k], np.float32(1e-30))[:, None]
        r = r + (sgn[:, None, None] * uk[:, :, None]) * vcol[k][:, None, :]

    rmx = jnp.sum(r * mx[:, None, :], axis=2)
    t_out = my[:, None, :] - rmx[:, :, None]
    r_ref[...] = r
    t_ref[...] = t_out


def _procrustes(weight, src, y):
    bb, n = weight.shape
    return pl.pallas_call(
        _finish_body,
        grid=(1,),
        in_specs=[
            pl.BlockSpec((bb, n), lambda i: (i * 0, i * 0)),
            pl.BlockSpec((bb, 3, n), lambda i: (i * 0, i * 0, i * 0)),
            pl.BlockSpec((bb, 3, n), lambda i: (i * 0, i * 0, i * 0)),
        ],
        out_specs=[
            pl.BlockSpec((bb, 3, 3), lambda i: (i * 0, i * 0, i * 0)),
            pl.BlockSpec((bb, 3, 3), lambda i: (i * 0, i * 0, i * 0)),
        ],
        out_shape=[
            jax.ShapeDtypeStruct((bb, 3, 3), jnp.float32),
            jax.ShapeDtypeStruct((bb, 3, 3), jnp.float32),
        ],
    )(weight, src, y)


def kernel(src_embedding, tgt_embedding, src, tgt, temperature, is_corr):
    with jax.enable_x64(False):
        weight, corres = _max_argmax(src_embedding, tgt_embedding, temperature)
        y = _sc_gather(tgt, corres)
        r, t = _procrustes(weight, src, y)
    return r, t, corres.astype(jnp.int64)[..., None], weight[..., None]


# speedup vs baseline: 1.1296x; 1.0011x over previous
import math

import jax
import jax.numpy as jnp
import numpy as np
from jax import lax
from jax.experimental import pallas as pl
from jax.experimental.pallas import tpu as pltpu
from jax.experimental.pallas import tpu_sc as plsc

_ROWS = 1024
_I0 = np.int32(0)
_EPS = np.float32(1e-7)


def _scores_body(temp_ref, src_ref, tgt_ref, w_ref, idx_ref):
    b = pl.program_id(0)
    n = tgt_ref.shape[2]
    t = temp_ref[b, 0, 0]
    a = src_ref[0]
    bt = tgt_ref[0]
    scale = np.float32(math.sqrt(src_ref.shape[1]))
    s = lax.dot_general(a, bt, (((0,), (0,)), ((), ())),
                        preferred_element_type=jnp.float32)
    z = t * (s * np.float32(1.0 / float(scale)))
    m = jnp.max(z, axis=1, keepdims=True)
    e = jnp.exp(z - m)
    ssum = jnp.sum(e, axis=1, keepdims=True)
    w = np.float32(1.0) / ssum
    ii = lax.broadcasted_iota(jnp.int32, (1, n), 1).astype(jnp.float32)
    cand = jnp.where(e == np.float32(1.0), ii, np.float32(n))
    idx = jnp.min(cand, axis=1).astype(jnp.int32)
    w_ref[0, 0] = w[:, 0]
    idx_ref[0, 0] = idx


def _max_argmax(src_embedding, tgt_embedding, temperature):
    bb, d, n = src_embedding.shape
    nt = n // _ROWS
    grid = (bb, nt)
    w3, i3 = pl.pallas_call(
        _scores_body,
        grid=grid,
        in_specs=[
            pl.BlockSpec(memory_space=pltpu.SMEM),
            pl.BlockSpec((1, d, _ROWS), lambda b, r: (b, b * 0, r)),
            pl.BlockSpec((1, d, n), lambda b, r: (b, b * 0, b * 0)),
        ],
        out_specs=[
            pl.BlockSpec((1, 1, _ROWS), lambda b, r: (b * nt + r, b * 0, b * 0)),
            pl.BlockSpec((1, 1, _ROWS), lambda b, r: (b * nt + r, b * 0, b * 0)),
        ],
        out_shape=[
            jax.ShapeDtypeStruct((bb * nt, 1, _ROWS), jnp.float32),
            jax.ShapeDtypeStruct((bb * nt, 1, _ROWS), jnp.int32),
        ],
    )(temperature, src_embedding, tgt_embedding)
    return w3.reshape(bb, n), i3.reshape(bb, n)


def _gather_body(tgt_ref, cor_ref, y_ref, t0, t1, t2, idxv, y0, y1, y2):
    i32 = jnp.int32
    cid = lax.axis_index("c").astype(i32)
    sid = lax.axis_index("s").astype(i32)
    wid = sid * i32(2) + cid
    b = lax.shift_right_logical(wid, i32(2))
    q = lax.bitwise_and(wid, i32(3))
    chunk = 512
    start = q * i32(chunk)
    row = b * i32(3)
    pltpu.sync_copy(tgt_ref.at[row], t0)
    pltpu.sync_copy(tgt_ref.at[row + i32(1)], t1)
    pltpu.sync_copy(tgt_ref.at[row + i32(2)], t2)
    pltpu.sync_copy(cor_ref.at[b, pl.ds(start, chunk)], idxv)

    for i in range(chunk // 16):
        sl = pl.ds(i * 16, 16)
        idx16 = idxv[sl]
        y0[sl] = plsc.load_gather(t0, [idx16])
        y1[sl] = plsc.load_gather(t1, [idx16])
        y2[sl] = plsc.load_gather(t2, [idx16])
    pltpu.sync_copy(y0, y_ref.at[row, pl.ds(start, chunk)])
    pltpu.sync_copy(y1, y_ref.at[row + i32(1), pl.ds(start, chunk)])
    pltpu.sync_copy(y2, y_ref.at[row + i32(2), pl.ds(start, chunk)])


def _sc_gather(tgt, corres):
    bb, _, n = tgt.shape
    chunk = 512
    mesh = plsc.VectorSubcoreMesh(core_axis_name="c", subcore_axis_name="s")
    y2d = pl.kernel(
        _gather_body,
        out_type=jax.ShapeDtypeStruct((bb * 3, n), jnp.float32),
        mesh=mesh,
        compiler_params=pltpu.CompilerParams(needs_layout_passes=False),
        scratch_types=[
            pltpu.VMEM((n,), jnp.float32),
            pltpu.VMEM((n,), jnp.float32),
            pltpu.VMEM((n,), jnp.float32),
            pltpu.VMEM((chunk,), jnp.int32),
            pltpu.VMEM((chunk,), jnp.float32),
            pltpu.VMEM((chunk,), jnp.float32),
            pltpu.VMEM((chunk,), jnp.float32),
        ],
    )(tgt.reshape(bb * 3, n), corres)
    return y2d.reshape(bb, 3, n)


def _finish_body(w_ref, x_ref, y_ref, r_ref, t_ref):
    w = w_ref[...]
    x = x_ref[...]
    y = y_ref[...]
    bb = w.shape[0]

    tot = jnp.sum(jnp.abs(w), axis=1)
    pp = tot + _EPS
    wb = w[:, None, :]
    sx = jnp.sum(x * wb, axis=2)
    sy = jnp.sum(y * wb, axis=2)
    mx = sx / pp[:, None]
    my = sy / pp[:, None]
    corr = (np.float32(2.0) - tot / pp)[:, None]

    bcol = []
    for c in range(3):
        wx = (w * x[:, c, :])[:, None, :]
        syx_c = jnp.sum(y * wx, axis=2)
        bcol.append(syx_c / pp[:, None] - my * (mx[:, c:c + 1] * corr))

    det = (
        bcol[0][:, 0] * (bcol[1][:, 1] * bcol[2][:, 2] - bcol[1][:, 2] * bcol[2][:, 1])
        - bcol[1][:, 0] * (bcol[0][:, 1] * bcol[2][:, 2] - bcol[0][:, 2] * bcol[2][:, 1])
        + bcol[2][:, 0] * (bcol[0][:, 1] * bcol[1][:, 2] - bcol[0][:, 2] * bcol[1][:, 1])
    )

    lane = lax.broadcasted_iota(jnp.int32, (bb, 3), 1)
    vcol = [(lane == k).astype(jnp.float32) for k in range(3)]

    one = np.float32(1.0)
    for _ in range(7):
        for (ip, iq) in ((0, 1), (0, 2), (1, 2)):
            bp, bq = bcol[ip], bcol[iq]
            alpha = jnp.sum(bp * bp, axis=1)
            beta = jnp.sum(bq * bq, axis=1)
            gamma = jnp.sum(bp * bq, axis=1)
            g0 = gamma == 0.0
            gsafe = jnp.where(g0, one, gamma)
            tau = (beta - alpha) / (np.float32(2.0) * gsafe)
            rr = jnp.sqrt(one + tau * tau)
            tt = jnp.where(tau >= 0.0, one / (tau + rr), one / (tau - rr))
            cc = one / jnp.sqrt(one + tt * tt)
            ss = cc * tt
            cc = jnp.where(g0, one, cc)[:, None]
            ss = jnp.where(g0, np.float32(0.0), ss)[:, None]
            newp = cc * bp - ss * bq
            newq = ss * bp + cc * bq
            bcol[ip], bcol[iq] = newp, newq
            vp, vq = vcol[ip], vcol[iq]
            vcol[ip] = cc * vp - ss * vq
            vcol[iq] = ss * vp + cc * vq

    sig = [jnp.sqrt(jnp.sum(bcol[k] * bcol[k], axis=1)) for k in range(3)]
    is0 = (sig[0] <= sig[1]) & (sig[0] <= sig[2])
    is1 = (~is0) & (sig[1] <= sig[2])
    is2 = ~(is0 | is1)
    neg = det < 0.0
    flips = [is0 & neg, is1 & neg, is2 & neg]

    r = jnp.zeros((bb, 3, 3), jnp.float32)
    for k in range(3):
        sgn = jnp.where(flips[k], np.float32(-1.0), one)
        uk = bcol[k] / jnp.maximum(sig[k], np.float32(1e-30))[:, None]
        r = r + (sgn[:, None, None] * uk[:, :, None]) * vcol[k][:, None, :]

    rmx = jnp.sum(r * mx[:, None, :], axis=2)
    t_out = my[:, None, :] - rmx[:, :, None]
    r_ref[...] = r
    t_ref[...] = t_out


def _procrustes(weight, src, y):
    bb, n = weight.shape
    return pl.pallas_call(
        _finish_body,
        grid=(1,),
        in_specs=[
            pl.BlockSpec((bb, n), lambda i: (i * 0, i * 0)),
            pl.BlockSpec((bb, 3, n), lambda i: (i * 0, i * 0, i * 0)),
            pl.BlockSpec((bb, 3, n), lambda i: (i * 0, i * 0, i * 0)),
        ],
        out_specs=[
            pl.BlockSpec((bb, 3, 3), lambda i: (i * 0, i * 0, i * 0)),
            pl.BlockSpec((bb, 3, 3), lambda i: (i * 0, i * 0, i * 0)),
        ],
        out_shape=[
            jax.ShapeDtypeStruct((bb, 3, 3), jnp.float32),
            jax.ShapeDtypeStruct((bb, 3, 3), jnp.float32),
        ],
    )(weight, src, y)


def kernel(src_embedding, tgt_embedding, src, tgt, temperature, is_corr):
    with jax.enable_x64(False):
        weight, corres = _max_argmax(src_embedding, tgt_embedding, temperature)
        y = _sc_gather(tgt, corres)
        r, t = _procrustes(weight, src, y)
    return r, t, corres.astype(jnp.int64)[..., None], weight[..., None]


# speedup vs baseline: 1.1365x; 1.0061x over previous
import math

import jax
import jax.numpy as jnp
import numpy as np
from jax import lax
from jax.experimental import pallas as pl
from jax.experimental.pallas import tpu as pltpu
from jax.experimental.pallas import tpu_sc as plsc

_ROWS = 1024
_I0 = np.int32(0)
_EPS = np.float32(1e-7)


def _scores_body(temp_ref, src_ref, tgt_ref, w_ref, idx_ref):
    b = pl.program_id(0)
    n = tgt_ref.shape[2]
    t = temp_ref[b, 0, 0]
    a = src_ref[0]
    bt = tgt_ref[0]
    scale = np.float32(math.sqrt(src_ref.shape[1]))
    s = lax.dot_general(a, bt, (((0,), (0,)), ((), ())),
                        preferred_element_type=jnp.float32)
    z = s * (t * np.float32(1.0 / float(scale)))
    m = jnp.max(z, axis=1, keepdims=True)
    e = jnp.exp(z - m)
    ssum = jnp.sum(e, axis=1, keepdims=True)
    w = np.float32(1.0) / ssum
    ii = lax.broadcasted_iota(jnp.int32, (1, n), 1).astype(jnp.float32)
    cand = jnp.where(e == np.float32(1.0), ii, np.float32(n))
    idx = jnp.min(cand, axis=1).astype(jnp.int32)
    w_ref[0, 0] = w[:, 0]
    idx_ref[0, 0] = idx


def _max_argmax(src_embedding, tgt_embedding, temperature):
    bb, d, n = src_embedding.shape
    nt = n // _ROWS
    grid = (bb, nt)
    w3, i3 = pl.pallas_call(
        _scores_body,
        grid=grid,
        in_specs=[
            pl.BlockSpec(memory_space=pltpu.SMEM),
            pl.BlockSpec((1, d, _ROWS), lambda b, r: (b, b * 0, r)),
            pl.BlockSpec((1, d, n), lambda b, r: (b, b * 0, b * 0)),
        ],
        out_specs=[
            pl.BlockSpec((1, 1, _ROWS), lambda b, r: (b * nt + r, b * 0, b * 0)),
            pl.BlockSpec((1, 1, _ROWS), lambda b, r: (b * nt + r, b * 0, b * 0)),
        ],
        out_shape=[
            jax.ShapeDtypeStruct((bb * nt, 1, _ROWS), jnp.float32),
            jax.ShapeDtypeStruct((bb * nt, 1, _ROWS), jnp.int32),
        ],
    )(temperature, src_embedding, tgt_embedding)
    return w3.reshape(bb, n), i3.reshape(bb, n)


def _gather_body(tgt_ref, cor_ref, y_ref, t0, t1, t2, idxv, y0, y1, y2):
    i32 = jnp.int32
    cid = lax.axis_index("c").astype(i32)
    sid = lax.axis_index("s").astype(i32)
    wid = sid * i32(2) + cid
    b = lax.shift_right_logical(wid, i32(2))
    q = lax.bitwise_and(wid, i32(3))
    chunk = 512
    start = q * i32(chunk)
    row = b * i32(3)
    pltpu.sync_copy(tgt_ref.at[row], t0)
    pltpu.sync_copy(tgt_ref.at[row + i32(1)], t1)
    pltpu.sync_copy(tgt_ref.at[row + i32(2)], t2)
    pltpu.sync_copy(cor_ref.at[b, pl.ds(start, chunk)], idxv)

    for i in range(chunk // 16):
        sl = pl.ds(i * 16, 16)
        idx16 = idxv[sl]
        y0[sl] = plsc.load_gather(t0, [idx16])
        y1[sl] = plsc.load_gather(t1, [idx16])
        y2[sl] = plsc.load_gather(t2, [idx16])
    pltpu.sync_copy(y0, y_ref.at[row, pl.ds(start, chunk)])
    pltpu.sync_copy(y1, y_ref.at[row + i32(1), pl.ds(start, chunk)])
    pltpu.sync_copy(y2, y_ref.at[row + i32(2), pl.ds(start, chunk)])


def _sc_gather(tgt, corres):
    bb, _, n = tgt.shape
    chunk = 512
    mesh = plsc.VectorSubcoreMesh(core_axis_name="c", subcore_axis_name="s")
    y2d = pl.kernel(
        _gather_body,
        out_type=jax.ShapeDtypeStruct((bb * 3, n), jnp.float32),
        mesh=mesh,
        compiler_params=pltpu.CompilerParams(needs_layout_passes=False),
        scratch_types=[
            pltpu.VMEM((n,), jnp.float32),
            pltpu.VMEM((n,), jnp.float32),
            pltpu.VMEM((n,), jnp.float32),
            pltpu.VMEM((chunk,), jnp.int32),
            pltpu.VMEM((chunk,), jnp.float32),
            pltpu.VMEM((chunk,), jnp.float32),
            pltpu.VMEM((chunk,), jnp.float32),
        ],
    )(tgt.reshape(bb * 3, n), corres)
    return y2d.reshape(bb, 3, n)


def _finish_body(w_ref, x_ref, y_ref, r_ref, t_ref):
    w = w_ref[...]
    x = x_ref[...]
    y = y_ref[...]
    bb = w.shape[0]

    tot = jnp.sum(jnp.abs(w), axis=1)
    pp = tot + _EPS
    wb = w[:, None, :]
    sx = jnp.sum(x * wb, axis=2)
    sy = jnp.sum(y * wb, axis=2)
    mx = sx / pp[:, None]
    my = sy / pp[:, None]
    corr = (np.float32(2.0) - tot / pp)[:, None]

    bcol = []
    for c in range(3):
        wx = (w * x[:, c, :])[:, None, :]
        syx_c = jnp.sum(y * wx, axis=2)
        bcol.append(syx_c / pp[:, None] - my * (mx[:, c:c + 1] * corr))

    det = (
        bcol[0][:, 0] * (bcol[1][:, 1] * bcol[2][:, 2] - bcol[1][:, 2] * bcol[2][:, 1])
        - bcol[1][:, 0] * (bcol[0][:, 1] * bcol[2][:, 2] - bcol[0][:, 2] * bcol[2][:, 1])
        + bcol[2][:, 0] * (bcol[0][:, 1] * bcol[1][:, 2] - bcol[0][:, 2] * bcol[1][:, 1])
    )

    lane = lax.broadcasted_iota(jnp.int32, (bb, 3), 1)
    vcol = [(lane == k).astype(jnp.float32) for k in range(3)]

    one = np.float32(1.0)
    for _ in range(7):
        for (ip, iq) in ((0, 1), (0, 2), (1, 2)):
            bp, bq = bcol[ip], bcol[iq]
            alpha = jnp.sum(bp * bp, axis=1)
            beta = jnp.sum(bq * bq, axis=1)
            gamma = jnp.sum(bp * bq, axis=1)
            g0 = gamma == 0.0
            gsafe = jnp.where(g0, one, gamma)
            tau = (beta - alpha) / (np.float32(2.0) * gsafe)
            rr = jnp.sqrt(one + tau * tau)
            tt = jnp.where(tau >= 0.0, one / (tau + rr), one / (tau - rr))
            cc = one / jnp.sqrt(one + tt * tt)
            ss = cc * tt
            cc = jnp.where(g0, one, cc)[:, None]
            ss = jnp.where(g0, np.float32(0.0), ss)[:, None]
            newp = cc * bp - ss * bq
            newq = ss * bp + cc * bq
            bcol[ip], bcol[iq] = newp, newq
            vp, vq = vcol[ip], vcol[iq]
            vcol[ip] = cc * vp - ss * vq
            vcol[iq] = ss * vp + cc * vq

    sig = [jnp.sqrt(jnp.sum(bcol[k] * bcol[k], axis=1)) for k in range(3)]
    is0 = (sig[0] <= sig[1]) & (sig[0] <= sig[2])
    is1 = (~is0) & (sig[1] <= sig[2])
    is2 = ~(is0 | is1)
    neg = det < 0.0
    flips = [is0 & neg, is1 & neg, is2 & neg]

    r = jnp.zeros((bb, 3, 3), jnp.float32)
    for k in range(3):
        sgn = jnp.where(flips[k], np.float32(-1.0), one)
        uk = bcol[k] / jnp.maximum(sig[k], np.float32(1e-30))[:, None]
        r = r + (sgn[:, None, None] * uk[:, :, None]) * vcol[k][:, None, :]

    rmx = jnp.sum(r * mx[:, None, :], axis=2)
    t_out = my[:, None, :] - rmx[:, :, None]
    r_ref[...] = r
    t_ref[...] = t_out


def _procrustes(weight, src, y):
    bb, n = weight.shape
    return pl.pallas_call(
        _finish_body,
        grid=(1,),
        in_specs=[
            pl.BlockSpec((bb, n), lambda i: (i * 0, i * 0)),
            pl.BlockSpec((bb, 3, n), lambda i: (i * 0, i * 0, i * 0)),
            pl.BlockSpec((bb, 3, n), lambda i: (i * 0, i * 0, i * 0)),
        ],
        out_specs=[
            pl.BlockSpec((bb, 3, 3), lambda i: (i * 0, i * 0, i * 0)),
            pl.BlockSpec((bb, 3, 3), lambda i: (i * 0, i * 0, i * 0)),
        ],
        out_shape=[
            jax.ShapeDtypeStruct((bb, 3, 3), jnp.float32),
            jax.ShapeDtypeStruct((bb, 3, 3), jnp.float32),
        ],
    )(weight, src, y)


def kernel(src_embedding, tgt_embedding, src, tgt, temperature, is_corr):
    with jax.enable_x64(False):
        weight, corres = _max_argmax(src_embedding, tgt_embedding, temperature)
        y = _sc_gather(tgt, corres)
        r, t = _procrustes(weight, src, y)
    return r, t, corres.astype(jnp.int64)[..., None], weight[..., None]


# speedup vs baseline: 1.1402x; 1.0033x over previous
import math

import jax
import jax.numpy as jnp
import numpy as np
from jax import lax
from jax.experimental import pallas as pl
from jax.experimental.pallas import tpu as pltpu
from jax.experimental.pallas import tpu_sc as plsc

_ROWS = 1024
_I0 = np.int32(0)
_EPS = np.float32(1e-7)


def _scores_body(temp_ref, src_ref, tgt_ref, w_ref, idx_ref):
    b = pl.program_id(0)
    n = tgt_ref.shape[2]
    t = temp_ref[b, 0, 0]
    a = src_ref[0]
    bt = tgt_ref[0]
    scale = np.float32(math.sqrt(src_ref.shape[1]))
    s = lax.dot_general(a, bt, (((0,), (0,)), ((), ())),
                        preferred_element_type=jnp.float32)
    z = s * (t * np.float32(1.0 / float(scale)))
    m = jnp.max(z, axis=1, keepdims=True)
    e = jnp.exp(z - m)
    ssum = jnp.sum(e, axis=1, keepdims=True)
    w = np.float32(1.0) / ssum
    ii = lax.broadcasted_iota(jnp.int32, (1, n), 1).astype(jnp.float32)
    cand = jnp.where(e == np.float32(1.0), ii, np.float32(n))
    idx = jnp.min(cand, axis=1).astype(jnp.int32)
    w_ref[0, 0] = w[:, 0]
    idx_ref[0, 0] = idx


def _max_argmax(src_embedding, tgt_embedding, temperature):
    bb, d, n = src_embedding.shape
    nt = n // _ROWS
    grid = (bb, nt)
    w3, i3 = pl.pallas_call(
        _scores_body,
        grid=grid,
        compiler_params=pltpu.CompilerParams(
            dimension_semantics=("parallel", "parallel")),
        in_specs=[
            pl.BlockSpec(memory_space=pltpu.SMEM),
            pl.BlockSpec((1, d, _ROWS), lambda b, r: (b, b * 0, r)),
            pl.BlockSpec((1, d, n), lambda b, r: (b, b * 0, b * 0)),
        ],
        out_specs=[
            pl.BlockSpec((1, 1, _ROWS), lambda b, r: (b * nt + r, b * 0, b * 0)),
            pl.BlockSpec((1, 1, _ROWS), lambda b, r: (b * nt + r, b * 0, b * 0)),
        ],
        out_shape=[
            jax.ShapeDtypeStruct((bb * nt, 1, _ROWS), jnp.float32),
            jax.ShapeDtypeStruct((bb * nt, 1, _ROWS), jnp.int32),
        ],
    )(temperature, src_embedding, tgt_embedding)
    return w3.reshape(bb, n), i3.reshape(bb, n)


def _gather_body(tgt_ref, cor_ref, y_ref, t0, t1, t2, idxv, y0, y1, y2):
    i32 = jnp.int32
    cid = lax.axis_index("c").astype(i32)
    sid = lax.axis_index("s").astype(i32)
    wid = sid * i32(2) + cid
    b = lax.shift_right_logical(wid, i32(2))
    q = lax.bitwise_and(wid, i32(3))
    chunk = 512
    start = q * i32(chunk)
    row = b * i32(3)
    pltpu.sync_copy(tgt_ref.at[row], t0)
    pltpu.sync_copy(tgt_ref.at[row + i32(1)], t1)
    pltpu.sync_copy(tgt_ref.at[row + i32(2)], t2)
    pltpu.sync_copy(cor_ref.at[b, pl.ds(start, chunk)], idxv)

    for i in range(chunk // 16):
        sl = pl.ds(i * 16, 16)
        idx16 = idxv[sl]
        y0[sl] = plsc.load_gather(t0, [idx16])
        y1[sl] = plsc.load_gather(t1, [idx16])
        y2[sl] = plsc.load_gather(t2, [idx16])
    pltpu.sync_copy(y0, y_ref.at[row, pl.ds(start, chunk)])
    pltpu.sync_copy(y1, y_ref.at[row + i32(1), pl.ds(start, chunk)])
    pltpu.sync_copy(y2, y_ref.at[row + i32(2), pl.ds(start, chunk)])


def _sc_gather(tgt, corres):
    bb, _, n = tgt.shape
    chunk = 512
    mesh = plsc.VectorSubcoreMesh(core_axis_name="c", subcore_axis_name="s")
    y2d = pl.kernel(
        _gather_body,
        out_type=jax.ShapeDtypeStruct((bb * 3, n), jnp.float32),
        mesh=mesh,
        compiler_params=pltpu.CompilerParams(needs_layout_passes=False),
        scratch_types=[
            pltpu.VMEM((n,), jnp.float32),
            pltpu.VMEM((n,), jnp.float32),
            pltpu.VMEM((n,), jnp.float32),
            pltpu.VMEM((chunk,), jnp.int32),
            pltpu.VMEM((chunk,), jnp.float32),
            pltpu.VMEM((chunk,), jnp.float32),
            pltpu.VMEM((chunk,), jnp.float32),
        ],
    )(tgt.reshape(bb * 3, n), corres)
    return y2d.reshape(bb, 3, n)


def _finish_body(w_ref, x_ref, y_ref, r_ref, t_ref):
    w = w_ref[...]
    x = x_ref[...]
    y = y_ref[...]
    bb = w.shape[0]

    tot = jnp.sum(jnp.abs(w), axis=1)
    pp = tot + _EPS
    wb = w[:, None, :]
    sx = jnp.sum(x * wb, axis=2)
    sy = jnp.sum(y * wb, axis=2)
    mx = sx / pp[:, None]
    my = sy / pp[:, None]
    corr = (np.float32(2.0) - tot / pp)[:, None]

    bcol = []
    for c in range(3):
        wx = (w * x[:, c, :])[:, None, :]
        syx_c = jnp.sum(y * wx, axis=2)
        bcol.append(syx_c / pp[:, None] - my * (mx[:, c:c + 1] * corr))

    det = (
        bcol[0][:, 0] * (bcol[1][:, 1] * bcol[2][:, 2] - bcol[1][:, 2] * bcol[2][:, 1])
        - bcol[1][:, 0] * (bcol[0][:, 1] * bcol[2][:, 2] - bcol[0][:, 2] * bcol[2][:, 1])
        + bcol[2][:, 0] * (bcol[0][:, 1] * bcol[1][:, 2] - bcol[0][:, 2] * bcol[1][:, 1])
    )

    lane = lax.broadcasted_iota(jnp.int32, (bb, 3), 1)
    vcol = [(lane == k).astype(jnp.float32) for k in range(3)]

    one = np.float32(1.0)
    for _ in range(7):
        for (ip, iq) in ((0, 1), (0, 2), (1, 2)):
            bp, bq = bcol[ip], bcol[iq]
            alpha = jnp.sum(bp * bp, axis=1)
            beta = jnp.sum(bq * bq, axis=1)
            gamma = jnp.sum(bp * bq, axis=1)
            g0 = gamma == 0.0
            gsafe = jnp.where(g0, one, gamma)
            tau = (beta - alpha) / (np.float32(2.0) * gsafe)
            rr = jnp.sqrt(one + tau * tau)
            tt = jnp.where(tau >= 0.0, one / (tau + rr), one / (tau - rr))
            cc = one / jnp.sqrt(one + tt * tt)
            ss = cc * tt
            cc = jnp.where(g0, one, cc)[:, None]
            ss = jnp.where(g0, np.float32(0.0), ss)[:, None]
            newp = cc * bp - ss * bq
            newq = ss * bp + cc * bq
            bcol[ip], bcol[iq] = newp, newq
            vp, vq = vcol[ip], vcol[iq]
            vcol[ip] = cc * vp - ss * vq
            vcol[iq] = ss * vp + cc * vq

    sig = [jnp.sqrt(jnp.sum(bcol[k] * bcol[k], axis=1)) for k in range(3)]
    is0 = (sig[0] <= sig[1]) & (sig[0] <= sig[2])
    is1 = (~is0) & (sig[1] <= sig[2])
    is2 = ~(is0 | is1)
    neg = det < 0.0
    flips = [is0 & neg, is1 & neg, is2 & neg]

    r = jnp.zeros((bb, 3, 3), jnp.float32)
    for k in range(3):
        sgn = jnp.where(flips[k], np.float32(-1.0), one)
        uk = bcol[k] / jnp.maximum(sig[k], np.float32(1e-30))[:, None]
        r = r + (sgn[:, None, None] * uk[:, :, None]) * vcol[k][:, None, :]

    rmx = jnp.sum(r * mx[:, None, :], axis=2)
    t_out = my[:, None, :] - rmx[:, :, None]
    r_ref[...] = r
    t_ref[...] = t_out


def _procrustes(weight, src, y):
    bb, n = weight.shape
    return pl.pallas_call(
        _finish_body,
        grid=(1,),
        in_specs=[
            pl.BlockSpec((bb, n), lambda i: (i * 0, i * 0)),
            pl.BlockSpec((bb, 3, n), lambda i: (i * 0, i * 0, i * 0)),
            pl.BlockSpec((bb, 3, n), lambda i: (i * 0, i * 0, i * 0)),
        ],
        out_specs=[
            pl.BlockSpec((bb, 3, 3), lambda i: (i * 0, i * 0, i * 0)),
            pl.BlockSpec((bb, 3, 3), lambda i: (i * 0, i * 0, i * 0)),
        ],
        out_shape=[
            jax.ShapeDtypeStruct((bb, 3, 3), jnp.float32),
            jax.ShapeDtypeStruct((bb, 3, 3), jnp.float32),
        ],
    )(weight, src, y)


def kernel(src_embedding, tgt_embedding, src, tgt, temperature, is_corr):
    with jax.enable_x64(False):
        weight, corres = _max_argmax(src_embedding, tgt_embedding, temperature)
        y = _sc_gather(tgt, corres)
        r, t = _procrustes(weight, src, y)
    return r, t, corres.astype(jnp.int64)[..., None], weight[..., None]
